# Initial kernel scaffold; baseline (speedup 1.0000x reference)
import jax
import jax.numpy as jnp
from jax.experimental import pallas as pl


def kernel(H_l, hop_attention, theta_out, theta_in, Theta, out_degree, in_degree, edge_weight, edge_index):
    raise NotImplementedError("write your pallas kernel here")


# speedup vs baseline: 1.2407x; 1.2407x over previous
import functools

import jax
import jax.numpy as jnp
from jax.experimental import pallas as pl
from jax.experimental.pallas import tpu as pltpu

N = 10000
E = 320000
D = 128

_ROWS_PER_BLOCK = 400
_N_BLOCKS = N // _ROWS_PER_BLOCK


def _prescale_body(h_ref, do_ref, di_ref, xo_ref, xi_ref):
    h = h_ref[...]
    xo_ref[...] = do_ref[...] * h
    xi_ref[...] = di_ref[...] * h


def _prescale(H, out_degree, in_degree):
    return pl.pallas_call(
        _prescale_body,
        grid=(_N_BLOCKS,),
        in_specs=[
            pl.BlockSpec((_ROWS_PER_BLOCK, D), lambda i: (i, 0)),
            pl.BlockSpec((_ROWS_PER_BLOCK, 1), lambda i: (i, 0)),
            pl.BlockSpec((_ROWS_PER_BLOCK, 1), lambda i: (i, 0)),
        ],
        out_specs=[
            pl.BlockSpec((_ROWS_PER_BLOCK, D), lambda i: (i, 0)),
            pl.BlockSpec((_ROWS_PER_BLOCK, D), lambda i: (i, 0)),
        ],
        out_shape=[
            jax.ShapeDtypeStruct((N, D), jnp.float32),
            jax.ShapeDtypeStruct((N, D), jnp.float32),
        ],
    )(H, out_degree[:, None], in_degree[:, None])


def _final_body(coef_ref, y1o_ref, y1i_ref, y2o_ref, y2i_ref, theta_ref,
                h_ref, out_ref):
    s = (coef_ref[0] * y1o_ref[...] + coef_ref[1] * y1i_ref[...]
         + coef_ref[2] * y2o_ref[...] + coef_ref[3] * y2i_ref[...])
    z = jnp.dot(s, theta_ref[...], preferred_element_type=jnp.float32)
    out_ref[...] = jax.nn.sigmoid(z) + h_ref[...]


def _final_stage(coefs, y1o, y1i, y2o, y2i, Theta, H):
    blk = pl.BlockSpec((_ROWS_PER_BLOCK, D), lambda i: (i, 0))
    return pl.pallas_call(
        _final_body,
        grid=(_N_BLOCKS,),
        in_specs=[
            pl.BlockSpec(memory_space=pltpu.SMEM),
            blk, blk, blk, blk,
            pl.BlockSpec((D, D), lambda i: (0, 0)),
            blk,
        ],
        out_specs=pl.BlockSpec((_ROWS_PER_BLOCK, D), lambda i: (i, 0)),
        out_shape=jax.ShapeDtypeStruct((N, D), jnp.float32),
    )(coefs, y1o, y1i, y2o, y2i, Theta, H)


def kernel(H_l, hop_attention, theta_out, theta_in, Theta, out_degree,
           in_degree, edge_weight, edge_index):
    row = edge_index[0]
    col = edge_index[1]

    alpha = jax.nn.softmax(hop_attention, axis=0)
    coefs = jnp.concatenate([
        (alpha[0] * theta_out[0])[None], (alpha[0] * theta_in[0])[None],
        (alpha[1] * theta_out[1])[None], (alpha[1] * theta_in[1])[None],
    ]).astype(jnp.float32)

    x_out, x_in = _prescale(H_l, out_degree, in_degree)

    def spmm(X):
        return jax.ops.segment_sum(X[col], row, num_segments=N)

    def spmm_T(X):
        return jax.ops.segment_sum(X[row], col, num_segments=N)

    y1o = spmm(x_out)
    y1i = spmm_T(x_in)
    y2o = spmm(y1o)
    y2i = spmm_T(y1i)

    return _final_stage(coefs, y1o, y1i, y2o, y2i, Theta, H_l)


# speedup vs baseline: 3.5547x; 2.8651x over previous
import functools

import jax
import jax.numpy as jnp
from jax import lax
from jax.experimental import pallas as pl
from jax.experimental.pallas import tpu as pltpu
from jax.experimental.pallas import tpu_sc as plsc

N = 10000
E = 320000
D = 128

_NP = 10112
_STRIPE = _NP // 16
_CHUNK = 128
_T = 160
_EPT = _CHUNK * _T
_EP = _EPT * 16
_DUMP = N
_NSLOT = 2
_KI = 32

_ROWS_PER_BLOCK = 400
_N_BLOCKS = N // _ROWS_PER_BLOCK


def _prescale_body(h_ref, do_ref, di_ref, xo_ref, xi_ref):
    h = h_ref[...]
    xo_ref[...] = do_ref[...] * h
    xi_ref[...] = di_ref[...] * h


def _prescale(H, out_degree, in_degree):
    return pl.pallas_call(
        _prescale_body,
        grid=(_N_BLOCKS,),
        in_specs=[
            pl.BlockSpec((_ROWS_PER_BLOCK, D), lambda i: (i, 0)),
            pl.BlockSpec((_ROWS_PER_BLOCK, 1), lambda i: (i, 0)),
            pl.BlockSpec((_ROWS_PER_BLOCK, 1), lambda i: (i, 0)),
        ],
        out_specs=[
            pl.BlockSpec((_ROWS_PER_BLOCK, D), lambda i: (i, 0)),
            pl.BlockSpec((_ROWS_PER_BLOCK, D), lambda i: (i, 0)),
        ],
        out_shape=[
            jax.ShapeDtypeStruct((_NP, D), jnp.float32),
            jax.ShapeDtypeStruct((_NP, D), jnp.float32),
        ],
    )(H, out_degree[:, None], in_degree[:, None])


def _final_body(coef_ref, y1o_ref, y1i_ref, y2o_ref, y2i_ref, theta_ref,
                h_ref, out_ref):
    s = (coef_ref[0] * y1o_ref[...] + coef_ref[1] * y1i_ref[...]
         + coef_ref[2] * y2o_ref[...] + coef_ref[3] * y2i_ref[...])
    z = jnp.dot(s, theta_ref[...], preferred_element_type=jnp.float32)
    out_ref[...] = jax.nn.sigmoid(z) + h_ref[...]


def _final_stage(coefs, y1o, y1i, y2o, y2i, Theta, H):
    blk = pl.BlockSpec((_ROWS_PER_BLOCK, D), lambda i: (i, 0))
    return pl.pallas_call(
        _final_body,
        grid=(_N_BLOCKS,),
        in_specs=[
            pl.BlockSpec(memory_space=pltpu.SMEM),
            blk, blk, blk, blk,
            pl.BlockSpec((D, D), lambda i: (0, 0)),
            blk,
        ],
        out_specs=blk,
        out_shape=jax.ShapeDtypeStruct((N, D), jnp.float32),
    )(coefs, y1o, y1i, y2o, y2i, Theta, H)


def _hop(src_hbm, gidx_hbm, sidx_hbm, idx_base, gbuf, sbuf, rbuf, acc,
         sgs, sss):
    def gather(j, b):
        return pltpu.async_copy(src_hbm.at[gbuf.at[j]], rbuf.at[b], sgs[b])

    def scatter(j, b):
        return pltpu.async_copy(rbuf.at[b], acc.at[sbuf.at[j]], sss[b],
                                add=True)

    @pl.loop(0, _T // _KI)
    def _blk(bi):
        rbase = idx_base + bi * _KI
        pltpu.sync_copy(gidx_hbm.at[pl.ds(rbase, _KI)], gbuf)
        pltpu.sync_copy(sidx_hbm.at[pl.ds(rbase, _KI)], sbuf)

        @pl.loop(0, _KI, step=_NSLOT)
        def _pair(j0):
            dg0 = gather(j0, 0)
            dg1 = gather(j0 + 1, 1)
            dg0.wait()
            ds0 = scatter(j0, 0)
            dg1.wait()
            ds1 = scatter(j0 + 1, 1)
            ds0.wait()
            ds1.wait()


def _sc_hops(x_out, x_in, gidx2, sidx2, zeros128):
    mesh = plsc.VectorSubcoreMesh(core_axis_name="c", subcore_axis_name="s")
    y_ty = jax.ShapeDtypeStruct((_NP, D), jnp.float32)

    @functools.partial(
        pl.kernel,
        out_type=[y_ty, y_ty, y_ty, y_ty],
        mesh=mesh,
        scratch_types=[
            pltpu.VMEM((_KI, _CHUNK), jnp.int32),
            pltpu.VMEM((_KI, _CHUNK), jnp.int32),
            pltpu.VMEM((_NSLOT, _CHUNK, D), jnp.float32),
            pltpu.VMEM_SHARED((_NP, D), jnp.float32),
        ] + [pltpu.SemaphoreType.DMA] * (2 * _NSLOT),
    )
    def k(xo_hbm, xi_hbm, gidx_hbm, sidx_hbm, z_hbm,
          y1o_hbm, y1i_hbm, y2o_hbm, y2i_hbm,
          gbuf, sbuf, rbuf, acc, *sems):
        sgs = sems[:_NSLOT]
        sss = sems[_NSLOT:]
        c = lax.axis_index("c")
        s = lax.axis_index("s")
        idx_base = c * (_EP // _CHUNK) + s * _T
        base = s * _STRIPE

        def zero_stripe():
            pltpu.sync_copy(z_hbm, rbuf.at[0])
            for p in range(_STRIPE // _CHUNK):
                pltpu.sync_copy(rbuf.at[0],
                                acc.at[pl.ds(base + p * _CHUNK, _CHUNK)])
            rem = _STRIPE % _CHUNK
            if rem:
                pltpu.sync_copy(rbuf.at[0, pl.ds(0, rem)],
                                acc.at[pl.ds(base + _STRIPE - rem, rem)])

        def write_out(y_hbm):
            for p in range(_STRIPE // _CHUNK):
                pltpu.sync_copy(acc.at[pl.ds(base + p * _CHUNK, _CHUNK)],
                                rbuf.at[0])
                pltpu.sync_copy(rbuf.at[0],
                                y_hbm.at[pl.ds(base + p * _CHUNK, _CHUNK)])
            rem = _STRIPE % _CHUNK
            if rem:
                off = _STRIPE - rem
                pltpu.sync_copy(acc.at[pl.ds(base + off, rem)],
                                rbuf.at[0, pl.ds(0, rem)])
                pltpu.sync_copy(rbuf.at[0, pl.ds(0, rem)],
                                y_hbm.at[pl.ds(base + off, rem)])

        def chain(x_hbm, y1_hbm, y2_hbm):
            zero_stripe()
            plsc.subcore_barrier()
            _hop(x_hbm, gidx_hbm, sidx_hbm, idx_base, gbuf, sbuf, rbuf,
                 acc, sgs, sss)
            plsc.subcore_barrier()
            write_out(y1_hbm)
            zero_stripe()
            plsc.subcore_barrier()
            _hop(y1_hbm, gidx_hbm, sidx_hbm, idx_base, gbuf, sbuf, rbuf,
                 acc, sgs, sss)
            plsc.subcore_barrier()
            write_out(y2_hbm)

        @pl.when(c == 0)
        def _():
            chain(xo_hbm, y1o_hbm, y2o_hbm)

        @pl.when(c == 1)
        def _():
            chain(xi_hbm, y1i_hbm, y2i_hbm)

    return k(x_out, x_in, gidx2, sidx2, zeros128)


def kernel(H_l, hop_attention, theta_out, theta_in, Theta, out_degree,
           in_degree, edge_weight, edge_index):
    row = edge_index[0]
    col = edge_index[1]

    alpha = jax.nn.softmax(hop_attention, axis=0)
    coefs = jnp.concatenate([
        (alpha[0] * theta_out[0])[None], (alpha[0] * theta_in[0])[None],
        (alpha[1] * theta_out[1])[None], (alpha[1] * theta_in[1])[None],
    ]).astype(jnp.float32)

    x_out, x_in = _prescale(H_l, out_degree, in_degree)

    pad = _EP - E
    g0 = jnp.pad(col, (0, pad))
    g1 = jnp.pad(row, (0, pad))
    s0 = jnp.pad(row, (0, pad), constant_values=_DUMP)
    s1 = jnp.pad(col, (0, pad), constant_values=_DUMP)
    gidx2 = jnp.concatenate([g0, g1]).reshape(2 * _EP // _CHUNK, _CHUNK)
    sidx2 = jnp.concatenate([s0, s1]).reshape(2 * _EP // _CHUNK, _CHUNK)

    zeros128 = jnp.zeros((_CHUNK, D), jnp.float32)

    y1o, y1i, y2o, y2i = _sc_hops(x_out, x_in, gidx2, sidx2, zeros128)
    return _final_stage(coefs, y1o, y1i, y2o, y2i, Theta, H_l)


# speedup vs baseline: 3.7171x; 1.0457x over previous
import functools

import jax
import jax.numpy as jnp
from jax import lax
from jax.experimental import pallas as pl
from jax.experimental.pallas import tpu as pltpu
from jax.experimental.pallas import tpu_sc as plsc

N = 10000
E = 320000
D = 128

_NP = 10112
_STRIPE = _NP // 16
_CHUNK = 128
_T = 160
_EPT = _CHUNK * _T
_EP = _EPT * 16
_DUMP = N
_NSLOT = 2
_KI = 32

_ROWS_PER_BLOCK = 400
_N_BLOCKS = N // _ROWS_PER_BLOCK


def _prescale_body(h_ref, do_ref, di_ref, xo_ref, xi_ref):
    h = h_ref[...]
    xo_ref[...] = do_ref[...] * h
    xi_ref[...] = di_ref[...] * h


def _prescale(H, out_degree, in_degree):
    return pl.pallas_call(
        _prescale_body,
        grid=(_N_BLOCKS,),
        in_specs=[
            pl.BlockSpec((_ROWS_PER_BLOCK, D), lambda i: (i, 0)),
            pl.BlockSpec((_ROWS_PER_BLOCK, 1), lambda i: (i, 0)),
            pl.BlockSpec((_ROWS_PER_BLOCK, 1), lambda i: (i, 0)),
        ],
        out_specs=[
            pl.BlockSpec((_ROWS_PER_BLOCK, D), lambda i: (i, 0)),
            pl.BlockSpec((_ROWS_PER_BLOCK, D), lambda i: (i, 0)),
        ],
        out_shape=[
            jax.ShapeDtypeStruct((_NP, D), jnp.float32),
            jax.ShapeDtypeStruct((_NP, D), jnp.float32),
        ],
    )(H, out_degree[:, None], in_degree[:, None])


def _final_body(coef_ref, y1o_ref, y1i_ref, y2o_ref, y2i_ref, theta_ref,
                h_ref, out_ref):
    s = (coef_ref[0] * y1o_ref[...] + coef_ref[1] * y1i_ref[...]
         + coef_ref[2] * y2o_ref[...] + coef_ref[3] * y2i_ref[...])
    z = jnp.dot(s, theta_ref[...], preferred_element_type=jnp.float32)
    out_ref[...] = jax.nn.sigmoid(z) + h_ref[...]


def _final_stage(coefs, y1o, y1i, y2o, y2i, Theta, H):
    blk = pl.BlockSpec((_ROWS_PER_BLOCK, D), lambda i: (i, 0))
    return pl.pallas_call(
        _final_body,
        grid=(_N_BLOCKS,),
        in_specs=[
            pl.BlockSpec(memory_space=pltpu.SMEM),
            blk, blk, blk, blk,
            pl.BlockSpec((D, D), lambda i: (0, 0)),
            blk,
        ],
        out_specs=blk,
        out_shape=jax.ShapeDtypeStruct((N, D), jnp.float32),
    )(coefs, y1o, y1i, y2o, y2i, Theta, H)


def _hop(src_hbm, gidx_hbm, sidx_hbm, idx_base, gbuf, sbuf, rbuf, acc,
         sgs, sss):
    def gather(j, b):
        return pltpu.async_copy(src_hbm.at[gbuf.at[j]], rbuf.at[b], sgs[b])

    def scatter(j, b):
        return pltpu.async_copy(rbuf.at[b], acc.at[sbuf.at[j]], sss[b],
                                add=True)

    def wait_gather(b):
        pltpu.make_async_copy(src_hbm.at[gbuf.at[0]], rbuf.at[b],
                              sgs[b]).wait()

    def wait_scatter(b):
        pltpu.make_async_copy(rbuf.at[b], acc.at[sbuf.at[0]],
                              sss[b]).wait()

    @pl.loop(0, _T // _KI)
    def _blk(bi):
        rbase = idx_base + bi * _KI
        pltpu.sync_copy(gidx_hbm.at[pl.ds(rbase, _KI)], gbuf)
        pltpu.sync_copy(sidx_hbm.at[pl.ds(rbase, _KI)], sbuf)

        gather(0, 0)
        gather(1, 1)

        @pl.loop(0, _KI - _NSLOT, step=_NSLOT)
        def _pair(j0):
            wait_gather(0)
            scatter(j0, 0)
            wait_gather(1)
            scatter(j0 + 1, 1)
            wait_scatter(0)
            gather(j0 + 2, 0)
            wait_scatter(1)
            gather(j0 + 3, 1)

        wait_gather(0)
        scatter(_KI - 2, 0)
        wait_gather(1)
        scatter(_KI - 1, 1)
        wait_scatter(0)
        wait_scatter(1)


def _sc_hops(x_out, x_in, gidx2, sidx2, zeros128):
    mesh = plsc.VectorSubcoreMesh(core_axis_name="c", subcore_axis_name="s")
    y_ty = jax.ShapeDtypeStruct((_NP, D), jnp.float32)

    @functools.partial(
        pl.kernel,
        out_type=[y_ty, y_ty, y_ty, y_ty],
        mesh=mesh,
        scratch_types=[
            pltpu.VMEM((_KI, _CHUNK), jnp.int32),
            pltpu.VMEM((_KI, _CHUNK), jnp.int32),
            pltpu.VMEM((_NSLOT, _CHUNK, D), jnp.float32),
            pltpu.VMEM_SHARED((_NP, D), jnp.float32),
        ] + [pltpu.SemaphoreType.DMA] * (2 * _NSLOT),
    )
    def k(xo_hbm, xi_hbm, gidx_hbm, sidx_hbm, z_hbm,
          y1o_hbm, y1i_hbm, y2o_hbm, y2i_hbm,
          gbuf, sbuf, rbuf, acc, *sems):
        sgs = sems[:_NSLOT]
        sss = sems[_NSLOT:]
        c = lax.axis_index("c")
        s = lax.axis_index("s")
        idx_base = c * (_EP // _CHUNK) + s * _T
        base = s * _STRIPE

        def zero_stripe():
            pltpu.sync_copy(z_hbm, rbuf.at[0])
            for p in range(_STRIPE // _CHUNK):
                pltpu.sync_copy(rbuf.at[0],
                                acc.at[pl.ds(base + p * _CHUNK, _CHUNK)])
            rem = _STRIPE % _CHUNK
            if rem:
                pltpu.sync_copy(rbuf.at[0, pl.ds(0, rem)],
                                acc.at[pl.ds(base + _STRIPE - rem, rem)])

        def write_out(y_hbm):
            for p in range(_STRIPE // _CHUNK):
                pltpu.sync_copy(acc.at[pl.ds(base + p * _CHUNK, _CHUNK)],
                                rbuf.at[0])
                pltpu.sync_copy(rbuf.at[0],
                                y_hbm.at[pl.ds(base + p * _CHUNK, _CHUNK)])
            rem = _STRIPE % _CHUNK
            if rem:
                off = _STRIPE - rem
                pltpu.sync_copy(acc.at[pl.ds(base + off, rem)],
                                rbuf.at[0, pl.ds(0, rem)])
                pltpu.sync_copy(rbuf.at[0, pl.ds(0, rem)],
                                y_hbm.at[pl.ds(base + off, rem)])

        def chain(x_hbm, y1_hbm, y2_hbm):
            zero_stripe()
            plsc.subcore_barrier()
            _hop(x_hbm, gidx_hbm, sidx_hbm, idx_base, gbuf, sbuf, rbuf,
                 acc, sgs, sss)
            plsc.subcore_barrier()
            write_out(y1_hbm)
            zero_stripe()
            plsc.subcore_barrier()
            _hop(y1_hbm, gidx_hbm, sidx_hbm, idx_base, gbuf, sbuf, rbuf,
                 acc, sgs, sss)
            plsc.subcore_barrier()
            write_out(y2_hbm)

        @pl.when(c == 0)
        def _():
            chain(xo_hbm, y1o_hbm, y2o_hbm)

        @pl.when(c == 1)
        def _():
            chain(xi_hbm, y1i_hbm, y2i_hbm)

    return k(x_out, x_in, gidx2, sidx2, zeros128)


def kernel(H_l, hop_attention, theta_out, theta_in, Theta, out_degree,
           in_degree, edge_weight, edge_index):
    row = edge_index[0]
    col = edge_index[1]

    alpha = jax.nn.softmax(hop_attention, axis=0)
    coefs = jnp.concatenate([
        (alpha[0] * theta_out[0])[None], (alpha[0] * theta_in[0])[None],
        (alpha[1] * theta_out[1])[None], (alpha[1] * theta_in[1])[None],
    ]).astype(jnp.float32)

    x_out, x_in = _prescale(H_l, out_degree, in_degree)

    pad = _EP - E
    g0 = jnp.pad(col, (0, pad))
    g1 = jnp.pad(row, (0, pad))
    s0 = jnp.pad(row, (0, pad), constant_values=_DUMP)
    s1 = jnp.pad(col, (0, pad), constant_values=_DUMP)
    gidx2 = jnp.concatenate([g0, g1]).reshape(2 * _EP // _CHUNK, _CHUNK)
    sidx2 = jnp.concatenate([s0, s1]).reshape(2 * _EP // _CHUNK, _CHUNK)

    zeros128 = jnp.zeros((_CHUNK, D), jnp.float32)

    y1o, y1i, y2o, y2i = _sc_hops(x_out, x_in, gidx2, sidx2, zeros128)
    return _final_stage(coefs, y1o, y1i, y2o, y2i, Theta, H_l)


# speedup vs baseline: 3.7438x; 1.0072x over previous
import functools

import jax
import jax.numpy as jnp
from jax import lax
from jax.experimental import pallas as pl
from jax.experimental.pallas import tpu as pltpu
from jax.experimental.pallas import tpu_sc as plsc

N = 10000
E = 320000
D = 128

_NP = 10112
_STRIPE = _NP // 16
_CHUNK = 128
_T = 160
_EPT = _CHUNK * _T
_EP = _EPT * 16
_DUMP = N
_NSLOT = 2
_KI = 32

_ROWS_PER_BLOCK = 400
_N_BLOCKS = N // _ROWS_PER_BLOCK


def _prescale_body(h_ref, do_ref, di_ref, xo_ref, xi_ref):
    h = h_ref[...]
    xo_ref[...] = do_ref[...] * h
    xi_ref[...] = di_ref[...] * h


def _prescale(H, out_degree, in_degree):
    return pl.pallas_call(
        _prescale_body,
        grid=(_N_BLOCKS,),
        in_specs=[
            pl.BlockSpec((_ROWS_PER_BLOCK, D), lambda i: (i, 0)),
            pl.BlockSpec((_ROWS_PER_BLOCK, 1), lambda i: (i, 0)),
            pl.BlockSpec((_ROWS_PER_BLOCK, 1), lambda i: (i, 0)),
        ],
        out_specs=[
            pl.BlockSpec((_ROWS_PER_BLOCK, D), lambda i: (i, 0)),
            pl.BlockSpec((_ROWS_PER_BLOCK, D), lambda i: (i, 0)),
        ],
        out_shape=[
            jax.ShapeDtypeStruct((_NP, D), jnp.float32),
            jax.ShapeDtypeStruct((_NP, D), jnp.float32),
        ],
    )(H, out_degree[:, None], in_degree[:, None])


def _final_body(coef_ref, y1o_ref, y1i_ref, y2o_ref, y2i_ref, theta_ref,
                h_ref, out_ref):
    s = (coef_ref[0] * y1o_ref[...] + coef_ref[1] * y1i_ref[...]
         + coef_ref[2] * y2o_ref[...] + coef_ref[3] * y2i_ref[...])
    z = jnp.dot(s, theta_ref[...], preferred_element_type=jnp.float32)
    out_ref[...] = jax.nn.sigmoid(z) + h_ref[...]


def _final_stage(coefs, y1o, y1i, y2o, y2i, Theta, H):
    blk = pl.BlockSpec((_ROWS_PER_BLOCK, D), lambda i: (i, 0))
    return pl.pallas_call(
        _final_body,
        grid=(_N_BLOCKS,),
        in_specs=[
            pl.BlockSpec(memory_space=pltpu.SMEM),
            blk, blk, blk, blk,
            pl.BlockSpec((D, D), lambda i: (0, 0)),
            blk,
        ],
        out_specs=blk,
        out_shape=jax.ShapeDtypeStruct((N, D), jnp.float32),
    )(coefs, y1o, y1i, y2o, y2i, Theta, H)


def _hop(src_hbm, gidx_hbm, sidx_hbm, idx_base, gbuf, sbuf, rbuf, acc,
         sgs, sss):
    def gather(j, b):
        pltpu.async_copy(src_hbm.at[gbuf.at[j, pl.ds(0, 64)]],
                         rbuf.at[b, pl.ds(0, 64)], sgs[2 * b])
        pltpu.async_copy(src_hbm.at[gbuf.at[j, pl.ds(64, 64)]],
                         rbuf.at[b, pl.ds(64, 64)], sgs[2 * b + 1])

    def scatter(j, b):
        return pltpu.async_copy(rbuf.at[b], acc.at[sbuf.at[j]], sss[b],
                                add=True)

    def wait_gather(b):
        pltpu.make_async_copy(src_hbm.at[gbuf.at[0, pl.ds(0, 64)]],
                              rbuf.at[b, pl.ds(0, 64)], sgs[2 * b]).wait()
        pltpu.make_async_copy(src_hbm.at[gbuf.at[0, pl.ds(0, 64)]],
                              rbuf.at[b, pl.ds(64, 64)],
                              sgs[2 * b + 1]).wait()

    def wait_scatter(b):
        pltpu.make_async_copy(rbuf.at[b], acc.at[sbuf.at[0]],
                              sss[b]).wait()

    @pl.loop(0, _T // _KI)
    def _blk(bi):
        rbase = idx_base + bi * _KI
        pltpu.sync_copy(gidx_hbm.at[pl.ds(rbase, _KI)], gbuf)
        pltpu.sync_copy(sidx_hbm.at[pl.ds(rbase, _KI)], sbuf)

        gather(0, 0)
        gather(1, 1)

        @pl.loop(0, _KI - _NSLOT, step=_NSLOT)
        def _pair(j0):
            wait_gather(0)
            scatter(j0, 0)
            wait_gather(1)
            scatter(j0 + 1, 1)
            wait_scatter(0)
            gather(j0 + 2, 0)
            wait_scatter(1)
            gather(j0 + 3, 1)

        wait_gather(0)
        scatter(_KI - 2, 0)
        wait_gather(1)
        scatter(_KI - 1, 1)
        wait_scatter(0)
        wait_scatter(1)


def _sc_hops(x_out, x_in, gidx2, sidx2, zeros128):
    mesh = plsc.VectorSubcoreMesh(core_axis_name="c", subcore_axis_name="s")
    y_ty = jax.ShapeDtypeStruct((_NP, D), jnp.float32)

    @functools.partial(
        pl.kernel,
        out_type=[y_ty, y_ty, y_ty, y_ty],
        mesh=mesh,
        scratch_types=[
            pltpu.VMEM((_KI, _CHUNK), jnp.int32),
            pltpu.VMEM((_KI, _CHUNK), jnp.int32),
            pltpu.VMEM((_NSLOT, _CHUNK, D), jnp.float32),
            pltpu.VMEM_SHARED((_NP, D), jnp.float32),
        ] + [pltpu.SemaphoreType.DMA] * (3 * _NSLOT),
    )
    def k(xo_hbm, xi_hbm, gidx_hbm, sidx_hbm, z_hbm,
          y1o_hbm, y1i_hbm, y2o_hbm, y2i_hbm,
          gbuf, sbuf, rbuf, acc, *sems):
        sgs = sems[:2 * _NSLOT]
        sss = sems[2 * _NSLOT:]
        c = lax.axis_index("c")
        s = lax.axis_index("s")
        idx_base = c * (_EP // _CHUNK) + s * _T
        base = s * _STRIPE

        def zero_stripe():
            pltpu.sync_copy(z_hbm, rbuf.at[0])
            for p in range(_STRIPE // _CHUNK):
                pltpu.sync_copy(rbuf.at[0],
                                acc.at[pl.ds(base + p * _CHUNK, _CHUNK)])
            rem = _STRIPE % _CHUNK
            if rem:
                pltpu.sync_copy(rbuf.at[0, pl.ds(0, rem)],
                                acc.at[pl.ds(base + _STRIPE - rem, rem)])

        def write_out(y_hbm):
            for p in range(_STRIPE // _CHUNK):
                pltpu.sync_copy(acc.at[pl.ds(base + p * _CHUNK, _CHUNK)],
                                rbuf.at[0])
                pltpu.sync_copy(rbuf.at[0],
                                y_hbm.at[pl.ds(base + p * _CHUNK, _CHUNK)])
            rem = _STRIPE % _CHUNK
            if rem:
                off = _STRIPE - rem
                pltpu.sync_copy(acc.at[pl.ds(base + off, rem)],
                                rbuf.at[0, pl.ds(0, rem)])
                pltpu.sync_copy(rbuf.at[0, pl.ds(0, rem)],
                                y_hbm.at[pl.ds(base + off, rem)])

        def chain(x_hbm, y1_hbm, y2_hbm):
            zero_stripe()
            plsc.subcore_barrier()
            _hop(x_hbm, gidx_hbm, sidx_hbm, idx_base, gbuf, sbuf, rbuf,
                 acc, sgs, sss)
            plsc.subcore_barrier()
            write_out(y1_hbm)
            zero_stripe()
            plsc.subcore_barrier()
            _hop(y1_hbm, gidx_hbm, sidx_hbm, idx_base, gbuf, sbuf, rbuf,
                 acc, sgs, sss)
            plsc.subcore_barrier()
            write_out(y2_hbm)

        @pl.when(c == 0)
        def _():
            chain(xo_hbm, y1o_hbm, y2o_hbm)

        @pl.when(c == 1)
        def _():
            chain(xi_hbm, y1i_hbm, y2i_hbm)

    return k(x_out, x_in, gidx2, sidx2, zeros128)


def kernel(H_l, hop_attention, theta_out, theta_in, Theta, out_degree,
           in_degree, edge_weight, edge_index):
    row = edge_index[0]
    col = edge_index[1]

    alpha = jax.nn.softmax(hop_attention, axis=0)
    coefs = jnp.concatenate([
        (alpha[0] * theta_out[0])[None], (alpha[0] * theta_in[0])[None],
        (alpha[1] * theta_out[1])[None], (alpha[1] * theta_in[1])[None],
    ]).astype(jnp.float32)

    x_out, x_in = _prescale(H_l, out_degree, in_degree)

    pad = _EP - E
    g0 = jnp.pad(col, (0, pad))
    g1 = jnp.pad(row, (0, pad))
    s0 = jnp.pad(row, (0, pad), constant_values=_DUMP)
    s1 = jnp.pad(col, (0, pad), constant_values=_DUMP)
    gidx2 = jnp.concatenate([g0, g1]).reshape(2 * _EP // _CHUNK, _CHUNK)
    sidx2 = jnp.concatenate([s0, s1]).reshape(2 * _EP // _CHUNK, _CHUNK)

    zeros128 = jnp.zeros((_CHUNK, D), jnp.float32)

    y1o, y1i, y2o, y2i = _sc_hops(x_out, x_in, gidx2, sidx2, zeros128)
    return _final_stage(coefs, y1o, y1i, y2o, y2i, Theta, H_l)
